# Initial kernel scaffold; baseline (speedup 1.0000x reference)
import jax
import jax.numpy as jnp
from jax.experimental import pallas as pl


def kernel(node_feat, edge_index, edge_feat, p, emb_h, We_e, be_e, Wq, Wk, Wv, Wpe, Wo, bo, bn1_g, bn1_b, W1, b1, W2, b2, bn2_g, bn2_b, Wr0, br0, Wr1, br1, Wr2, br2):
    raise NotImplementedError("write your pallas kernel here")


# speedup vs baseline: 15.6980x; 15.6980x over previous
import functools

import jax
import jax.numpy as jnp
from jax import lax
from jax.experimental import pallas as pl
from jax.experimental.pallas import tpu as pltpu
from jax.experimental.pallas import tpu_sc as plsc

N = 10000
E = 320000
D = 128
H = 8
DH = 16
L = 4
PAD_ATOM = 32

NC = 2
NS = 16
NW = NC * NS
EPW = E // NW
EC = 40
NCHUNK = EPW // EC
NACC = 10240
NPT = NACC // NS
NR = 40
NZ = NACC // 8
NZPT = NZ // NS
INV_SQRT_DH = 1.0 / (DH ** 0.5)
_GROUPS = ((0, 0), (16, 0), (24, 8))


def _embed_body(nf_ref, emb_ref, out_ref):
    nf = nf_ref[...]
    oh = (jnp.broadcast_to(nf, (N, PAD_ATOM))
          == lax.broadcasted_iota(jnp.int32, (N, PAD_ATOM), 1))
    out_ref[...] = jnp.dot(oh.astype(jnp.float32), emb_ref[...],
                           preferred_element_type=jnp.float32)


_embed_call = pl.pallas_call(
    _embed_body,
    out_shape=jax.ShapeDtypeStruct((N, D), jnp.float32),
)


def _qkv_body(h_ref, wkv_ref, wq_ref, kv_ref, q_ref):
    h = h_ref[...]
    kv_ref[...] = jnp.dot(h, wkv_ref[...], preferred_element_type=jnp.float32)
    q_ref[...] = jnp.dot(h, wq_ref[...], preferred_element_type=jnp.float32)


_qkv_call = pl.pallas_call(
    _qkv_body,
    out_shape=(jax.ShapeDtypeStruct((N, 2 * D), jnp.float32),
               jax.ShapeDtypeStruct((N, D), jnp.float32)),
)


def _tail_body(acc_ref, z_ref, h_ref, wo_ref, bo_ref, g1_ref, b1_ref, w1_ref,
               bb1_ref, w2_ref, bb2_ref, g2_ref, b2_ref, out_ref):
    wv = acc_ref[0, :N] + acc_ref[1, :N]
    z = z_ref[0] + z_ref[1]
    zinv = 1.0 / (z + 1e-6)
    row = lax.broadcasted_iota(jnp.int32, (H, D), 0)
    col = lax.broadcasted_iota(jnp.int32, (H, D), 1) // DH
    ex = (row == col).astype(jnp.float32)
    attn = wv * jnp.dot(zinv, ex, preferred_element_type=jnp.float32)
    h_att = jnp.dot(attn, wo_ref[...],
                    preferred_element_type=jnp.float32) + bo_ref[...]
    h1 = h_ref[...] + h_att
    mu = jnp.mean(h1, axis=0, keepdims=True)
    var = jnp.mean((h1 - mu) ** 2, axis=0, keepdims=True)
    h2 = (h1 - mu) * lax.rsqrt(var + 1e-5) * g1_ref[...] + b1_ref[...]
    r = jnp.maximum(jnp.dot(h2, w1_ref[...],
                            preferred_element_type=jnp.float32) + bb1_ref[...],
                    0.0)
    h3 = h2 + jnp.dot(r, w2_ref[...],
                      preferred_element_type=jnp.float32) + bb2_ref[...]
    mu2 = jnp.mean(h3, axis=0, keepdims=True)
    var2 = jnp.mean((h3 - mu2) ** 2, axis=0, keepdims=True)
    out_ref[...] = ((h3 - mu2) * lax.rsqrt(var2 + 1e-5) * g2_ref[...]
                    + b2_ref[...])


_tail_call = pl.pallas_call(
    _tail_body,
    out_shape=jax.ShapeDtypeStruct((N, D), jnp.float32),
)


def _readout_body(h_ref, w0_ref, b0_ref, w1_ref, b1_ref, w2_ref, b2_ref,
                  out_ref):
    x = jnp.maximum(jnp.dot(h_ref[...], w0_ref[...],
                            preferred_element_type=jnp.float32) + b0_ref[...],
                    0.0)
    x = jnp.maximum(jnp.dot(x, w1_ref[...],
                            preferred_element_type=jnp.float32) + b1_ref[...],
                    0.0)
    out_ref[...] = jnp.dot(x, w2_ref[...],
                           preferred_element_type=jnp.float32) + b2_ref[...]


_readout_call = pl.pallas_call(
    _readout_body,
    out_shape=jax.ShapeDtypeStruct((N, 1), jnp.float32),
)


def _edge_body(kv_hbm, q_hbm, src_hbm, dst_hbm, ef_hbm, epw_hbm, zrows_hbm,
               out_hbm, outz_hbm,
               sidx, didx, dzidx, kvb, qb, pb, pbz, efb, epb, zb,
               acc, accz, sem1, sem2):
    cid = lax.axis_index("c")
    sid = lax.axis_index("s")
    wid = cid * NS + sid

    pltpu.sync_copy(epw_hbm, epb)
    epv = [[epb[b, pl.ds(h * DH, DH)] for h in range(H)] for b in range(5)]

    pltpu.sync_copy(zrows_hbm, zb)
    base_row = sid * NPT
    for r in range(NPT // NR):
        pltpu.sync_copy(zb, acc.at[pl.ds(base_row + r * NR, NR)])
    for r in range(NZPT // NR):
        pltpu.sync_copy(zb, accz.at[pl.ds(sid * NZPT + r * NR, NR)])
    z16 = jnp.zeros((DH,), jnp.float32)

    def zinit_body(e, _):
        for g in range(8):
            pbz[e, pl.ds(g * DH, DH)] = z16
        return 0

    lax.fori_loop(0, EC, zinit_body, 0)
    plsc.subcore_barrier()

    lanes = lax.iota(jnp.int32, DH)
    ebase = wid * EPW

    def chunk_body(j, _):
        off = ebase + j * EC
        pltpu.sync_copy(src_hbm.at[pl.ds(off, EC)], sidx)
        pltpu.sync_copy(dst_hbm.at[pl.ds(off, EC)], didx)
        pltpu.sync_copy(ef_hbm.at[pl.ds(off, EC)], efb)
        cp1 = pltpu.async_copy(kv_hbm.at[sidx], kvb, sem1)
        cp2 = pltpu.async_copy(q_hbm.at[didx], qb, sem2)
        zoffs = {}
        for g0, l0 in _GROUPS:
            dv = didx[pl.ds(g0, DH)]
            dzidx[pl.ds(g0, DH)] = lax.shift_right_logical(dv, 3)
            offv = (dv & 7) * DH
            for i in range(l0, DH):
                zoffs[g0 + i] = offv[i]
        cp1.wait()
        cp2.wait()

        for g0, l0 in _GROUPS:
            for i in range(l0, DH):
                e = g0 + i
                efv = efb[e, pl.ds(0, 16)]
                ef0 = efv[0]
                ef1 = efv[1]
                ef2 = efv[2]
                ef3 = efv[3]
                zv = jnp.zeros((DH,), jnp.float32)
                for h in range(H):
                    kh = kvb[e, pl.ds(h * DH, DH)]
                    vh = kvb[e, pl.ds(D + h * DH, DH)]
                    qh = qb[e, pl.ds(h * DH, DH)]
                    eph = (epv[4][h] + ef0 * epv[0][h] + ef1 * epv[1][h]
                           + ef2 * epv[2][h] + ef3 * epv[3][h])
                    sc = jnp.sum(kh * qh * eph) * INV_SQRT_DH
                    sc = jnp.minimum(jnp.maximum(sc, -5.0), 5.0)
                    sv = jnp.exp(jnp.full((DH,), sc, jnp.float32))
                    pb[e, pl.ds(h * DH, DH)] = vh * sv
                    zv = jnp.where(lanes == h, sv, zv)
                pbz[e, pl.ds(zoffs[e], DH)] = zv

        pltpu.sync_copy(pb, acc.at[didx], add=True)
        pltpu.sync_copy(pbz, accz.at[dzidx], add=True)

        for e in range(EC):
            pbz[e, pl.ds(zoffs[e], DH)] = z16
        return 0

    lax.fori_loop(0, NCHUNK, chunk_body, 0)
    plsc.subcore_barrier()

    for r in range(NPT // NR):
        sl = pl.ds(base_row + r * NR, NR)
        pltpu.sync_copy(acc.at[sl], zb)
        pltpu.sync_copy(zb, out_hbm.at[cid, sl])
    for r in range(NZPT // NR):
        zsl = pl.ds(sid * NZPT + r * NR, NR)
        pltpu.sync_copy(accz.at[zsl], zb)
        pltpu.sync_copy(zb, outz_hbm.at[cid, zsl])


@functools.lru_cache(maxsize=1)
def _edge_call():
    return pl.kernel(
        _edge_body,
        out_type=(jax.ShapeDtypeStruct((NC, NACC, D), jnp.float32),
                  jax.ShapeDtypeStruct((NC, NZ, D), jnp.float32)),
        mesh=plsc.VectorSubcoreMesh(core_axis_name="c", subcore_axis_name="s",
                                    num_cores=NC, num_subcores=NS),
        compiler_params=pltpu.CompilerParams(needs_layout_passes=False),
        scratch_types=[
            pltpu.VMEM((EC,), jnp.int32),
            pltpu.VMEM((EC,), jnp.int32),
            pltpu.VMEM((EC,), jnp.int32),
            pltpu.VMEM((EC, 2 * D), jnp.float32),
            pltpu.VMEM((EC, D), jnp.float32),
            pltpu.VMEM((EC, D), jnp.float32),
            pltpu.VMEM((EC, D), jnp.float32),
            pltpu.VMEM((EC, 16), jnp.float32),
            pltpu.VMEM((5, D), jnp.float32),
            pltpu.VMEM((NR, D), jnp.float32),
            pltpu.VMEM_SHARED((NACC, D), jnp.float32),
            pltpu.VMEM_SHARED((NZ, D), jnp.float32),
            pltpu.SemaphoreType.DMA,
            pltpu.SemaphoreType.DMA,
        ],
    )


def kernel(node_feat, edge_index, edge_feat, p, emb_h, We_e, be_e, Wq, Wk,
           Wv, Wpe, Wo, bo, bn1_g, bn1_b, W1, b1, W2, b2, bn2_g, bn2_b,
           Wr0, br0, Wr1, br1, Wr2, br2):
    nf = node_feat.astype(jnp.int32).reshape(N, 1)
    ei = edge_index.astype(jnp.int32)
    emb_pad = jnp.zeros((PAD_ATOM, D), jnp.float32).at[:emb_h.shape[0]].set(emb_h)
    zrows = jnp.zeros((NR, D), jnp.float32)
    ef16 = jnp.zeros((E, 16), jnp.float32).at[:, :4].set(edge_feat)

    h = _embed_call(nf, emb_pad)
    for l in range(L):
        wkv = jnp.concatenate([Wk[l], Wv[l]], axis=1)
        kv, q = _qkv_call(h, wkv, Wq[l])
        epw = jnp.concatenate(
            [We_e @ Wpe[l], (be_e @ Wpe[l]).reshape(1, D)], axis=0)
        acc2, accz = _edge_call()(kv, q, ei[0], ei[1], ef16, epw, zrows)
        z2 = accz.reshape(NC, NZ, 8, 16)[:, :, :, :H].reshape(NC, NACC, H)[:, :N, :]
        h = _tail_call(acc2, z2, h, Wo[l], bo[l].reshape(1, D),
                       bn1_g[l].reshape(1, D), bn1_b[l].reshape(1, D),
                       W1[l], b1[l].reshape(1, 2 * D), W2[l],
                       b2[l].reshape(1, D), bn2_g[l].reshape(1, D),
                       bn2_b[l].reshape(1, D))
    return _readout_call(h, Wr0, br0.reshape(1, D // 2), Wr1,
                         br1.reshape(1, D // 4), Wr2, br2.reshape(1, 1))


# speedup vs baseline: 21.0353x; 1.3400x over previous
import functools

import jax
import jax.numpy as jnp
from jax import lax
from jax.experimental import pallas as pl
from jax.experimental.pallas import tpu as pltpu
from jax.experimental.pallas import tpu_sc as plsc

N = 10000
E = 320000
D = 128
H = 8
DH = 16
L = 4
PAD_ATOM = 32

NC = 2
NS = 16
NW = NC * NS
EPW = E // NW
EC = 40
NCHUNK = EPW // EC
NACC = 10240
NPT = NACC // NS
NR = 40
NZ = NACC // 8
NZPT = NZ // NS
INV_SQRT_DH = 1.0 / (DH ** 0.5)
_GROUPS = ((0, 0), (16, 0), (24, 8))
_PERM = tuple((j % 8) * DH + 2 * i + (j // 8)
              for i in range(H) for j in range(16))


def _embed_body(nf_ref, emb_ref, out_ref):
    nf = nf_ref[...]
    oh = (jnp.broadcast_to(nf, (N, PAD_ATOM))
          == lax.broadcasted_iota(jnp.int32, (N, PAD_ATOM), 1))
    out_ref[...] = jnp.dot(oh.astype(jnp.float32), emb_ref[...],
                           preferred_element_type=jnp.float32)


_embed_call = pl.pallas_call(
    _embed_body,
    out_shape=jax.ShapeDtypeStruct((N, D), jnp.float32),
)


def _qkv_body(h_ref, wkv_ref, wq_ref, kv_ref, q_ref):
    h = h_ref[...]
    kv_ref[...] = jnp.dot(h, wkv_ref[...], preferred_element_type=jnp.float32)
    q_ref[...] = jnp.dot(h, wq_ref[...], preferred_element_type=jnp.float32)


_qkv_call = pl.pallas_call(
    _qkv_body,
    out_shape=(jax.ShapeDtypeStruct((N, 2 * D), jnp.float32),
               jax.ShapeDtypeStruct((N, D), jnp.float32)),
)


EPB = 8000


def _ep_body(ef_ref, m_ref, b_ref, out_ref):
    out_ref[...] = jnp.dot(ef_ref[...], m_ref[...],
                           preferred_element_type=jnp.float32) + b_ref[...]


_ep_call = pl.pallas_call(
    _ep_body,
    grid=(E // EPB,),
    in_specs=[pl.BlockSpec((EPB, 16), lambda i: (i, 0)),
              pl.BlockSpec((16, D), lambda i: (0, 0)),
              pl.BlockSpec((1, D), lambda i: (0, 0))],
    out_specs=pl.BlockSpec((EPB, D), lambda i: (i, 0)),
    out_shape=jax.ShapeDtypeStruct((E, D), jnp.float32),
)


def _tail_body(acc_ref, z_ref, h_ref, wo_ref, bo_ref, g1_ref, b1_ref, w1_ref,
               bb1_ref, w2_ref, bb2_ref, g2_ref, b2_ref, out_ref):
    wv = acc_ref[0, :N] + acc_ref[1, :N]
    z = z_ref[0] + z_ref[1]
    zinv = 1.0 / (z + 1e-6)
    row = lax.broadcasted_iota(jnp.int32, (H, D), 0)
    col = lax.broadcasted_iota(jnp.int32, (H, D), 1) % H
    ex = (row == col).astype(jnp.float32)
    attn = wv * jnp.dot(zinv, ex, preferred_element_type=jnp.float32)
    h_att = jnp.dot(attn, wo_ref[...],
                    preferred_element_type=jnp.float32) + bo_ref[...]
    h1 = h_ref[...] + h_att
    mu = jnp.mean(h1, axis=0, keepdims=True)
    var = jnp.mean((h1 - mu) ** 2, axis=0, keepdims=True)
    h2 = (h1 - mu) * lax.rsqrt(var + 1e-5) * g1_ref[...] + b1_ref[...]
    r = jnp.maximum(jnp.dot(h2, w1_ref[...],
                            preferred_element_type=jnp.float32) + bb1_ref[...],
                    0.0)
    h3 = h2 + jnp.dot(r, w2_ref[...],
                      preferred_element_type=jnp.float32) + bb2_ref[...]
    mu2 = jnp.mean(h3, axis=0, keepdims=True)
    var2 = jnp.mean((h3 - mu2) ** 2, axis=0, keepdims=True)
    out_ref[...] = ((h3 - mu2) * lax.rsqrt(var2 + 1e-5) * g2_ref[...]
                    + b2_ref[...])


_tail_call = pl.pallas_call(
    _tail_body,
    out_shape=jax.ShapeDtypeStruct((N, D), jnp.float32),
)


def _readout_body(h_ref, w0_ref, b0_ref, w1_ref, b1_ref, w2_ref, b2_ref,
                  out_ref):
    x = jnp.maximum(jnp.dot(h_ref[...], w0_ref[...],
                            preferred_element_type=jnp.float32) + b0_ref[...],
                    0.0)
    x = jnp.maximum(jnp.dot(x, w1_ref[...],
                            preferred_element_type=jnp.float32) + b1_ref[...],
                    0.0)
    out_ref[...] = jnp.dot(x, w2_ref[...],
                           preferred_element_type=jnp.float32) + b2_ref[...]


_readout_call = pl.pallas_call(
    _readout_body,
    out_shape=jax.ShapeDtypeStruct((N, 1), jnp.float32),
)


def _edge_body(kv_hbm, q_hbm, src_hbm, dst_hbm, ep_hbm, zrows_hbm,
               out_hbm, outz_hbm,
               sidx, didx, dzidx, kvb, qb, eb, pb, pbz, zb,
               acc, accz, sem1, sem2):
    cid = lax.axis_index("c")
    sid = lax.axis_index("s")
    wid = cid * NS + sid

    pltpu.sync_copy(zrows_hbm, zb)
    base_row = sid * NPT
    for r in range(NPT // NR):
        pltpu.sync_copy(zb, acc.at[pl.ds(base_row + r * NR, NR)])
    for r in range(NZPT // NR):
        pltpu.sync_copy(zb, accz.at[pl.ds(sid * NZPT + r * NR, NR)])
    z16 = jnp.zeros((DH,), jnp.float32)

    def zinit_body(e, _):
        for g in range(8):
            pbz[e, pl.ds(g * DH, DH)] = z16
        return 0

    lax.fori_loop(0, EC, zinit_body, 0)
    plsc.subcore_barrier()

    iota16 = lax.iota(jnp.int32, DH)
    idx_swap = ((iota16 + 8) & 15).astype(jnp.int32)
    zmask = iota16 < 8
    ebase = wid * EPW

    def chunk_body(j, _):
        off = ebase + j * EC
        pltpu.sync_copy(src_hbm.at[pl.ds(off, EC)], sidx)
        pltpu.sync_copy(dst_hbm.at[pl.ds(off, EC)], didx)
        pltpu.sync_copy(ep_hbm.at[pl.ds(off, EC)], eb)
        cp1 = pltpu.async_copy(kv_hbm.at[sidx], kvb, sem1)
        cp2 = pltpu.async_copy(q_hbm.at[didx], qb, sem2)
        zoffs = {}
        for g0, l0 in _GROUPS:
            dv = didx[pl.ds(g0, DH)]
            dzidx[pl.ds(g0, DH)] = lax.shift_right_logical(dv, 3)
            offv = (dv & 7) * DH
            for i in range(l0, DH):
                zoffs[g0 + i] = offv[i]
        cp1.wait()
        cp2.wait()

        for e in range(EC):
            ts = []
            for b in range(H):
                kb = kvb[e, pl.ds(b * DH, DH)]
                qv = qb[e, pl.ds(b * DH, DH)]
                ev = eb[e, pl.ds(b * DH, DH)]
                ts.append(kb * qv * ev)
            while len(ts) > 1:
                ts = [ts[i] + ts[i + 1] for i in range(0, len(ts), 2)]
            t = ts[0]
            s = (t + t[idx_swap]) * INV_SQRT_DH
            s = jnp.exp(jnp.minimum(jnp.maximum(s, -5.0), 5.0))
            for b in range(H):
                vb = kvb[e, pl.ds(D + b * DH, DH)]
                pb[e, pl.ds(b * DH, DH)] = vb * s
            zv = jnp.where(zmask, s, 0.0)
            pbz[e, pl.ds(zoffs[e], DH)] = zv

        pltpu.sync_copy(pb, acc.at[didx], add=True)
        pltpu.sync_copy(pbz, accz.at[dzidx], add=True)

        for e in range(EC):
            pbz[e, pl.ds(zoffs[e], DH)] = z16
        return 0

    lax.fori_loop(0, NCHUNK, chunk_body, 0)
    plsc.subcore_barrier()

    for r in range(NPT // NR):
        sl = pl.ds(base_row + r * NR, NR)
        pltpu.sync_copy(acc.at[sl], zb)
        pltpu.sync_copy(zb, out_hbm.at[cid, sl])
    for r in range(NZPT // NR):
        zsl = pl.ds(sid * NZPT + r * NR, NR)
        pltpu.sync_copy(accz.at[zsl], zb)
        pltpu.sync_copy(zb, outz_hbm.at[cid, zsl])


@functools.lru_cache(maxsize=1)
def _edge_call():
    return pl.kernel(
        _edge_body,
        out_type=(jax.ShapeDtypeStruct((NC, NACC, D), jnp.float32),
                  jax.ShapeDtypeStruct((NC, NZ, D), jnp.float32)),
        mesh=plsc.VectorSubcoreMesh(core_axis_name="c", subcore_axis_name="s",
                                    num_cores=NC, num_subcores=NS),
        compiler_params=pltpu.CompilerParams(needs_layout_passes=False),
        scratch_types=[
            pltpu.VMEM((EC,), jnp.int32),
            pltpu.VMEM((EC,), jnp.int32),
            pltpu.VMEM((EC,), jnp.int32),
            pltpu.VMEM((EC, 2 * D), jnp.float32),
            pltpu.VMEM((EC, D), jnp.float32),
            pltpu.VMEM((EC, D), jnp.float32),
            pltpu.VMEM((EC, D), jnp.float32),
            pltpu.VMEM((EC, D), jnp.float32),
            pltpu.VMEM((NR, D), jnp.float32),
            pltpu.VMEM_SHARED((NACC, D), jnp.float32),
            pltpu.VMEM_SHARED((NZ, D), jnp.float32),
            pltpu.SemaphoreType.DMA,
            pltpu.SemaphoreType.DMA,
        ],
    )


def kernel(node_feat, edge_index, edge_feat, p, emb_h, We_e, be_e, Wq, Wk,
           Wv, Wpe, Wo, bo, bn1_g, bn1_b, W1, b1, W2, b2, bn2_g, bn2_b,
           Wr0, br0, Wr1, br1, Wr2, br2):
    nf = node_feat.astype(jnp.int32).reshape(N, 1)
    ei = edge_index.astype(jnp.int32)
    emb_pad = jnp.zeros((PAD_ATOM, D), jnp.float32).at[:emb_h.shape[0]].set(emb_h)
    zrows = jnp.zeros((NR, D), jnp.float32)
    ef16 = jnp.zeros((E, 16), jnp.float32).at[:, :4].set(edge_feat)

    perm = jnp.array(_PERM, jnp.int32)
    h = _embed_call(nf, emb_pad)
    for l in range(L):
        wkv = jnp.concatenate([Wk[l][:, perm], Wv[l][:, perm]], axis=1)
        kv, q = _qkv_call(h, wkv, Wq[l][:, perm])
        m16 = jnp.zeros((16, D), jnp.float32).at[:4].set(We_e @ Wpe[l])
        ept = _ep_call(ef16, m16[:, perm], (be_e @ Wpe[l])[perm].reshape(1, D))
        acc2, accz = _edge_call()(kv, q, ei[0], ei[1], ept, zrows)
        z2 = accz.reshape(NC, NZ, 8, 16)[:, :, :, :H].reshape(NC, NACC, H)[:, :N, :]
        h = _tail_call(acc2, z2, h, Wo[l][perm, :], bo[l].reshape(1, D),
                       bn1_g[l].reshape(1, D), bn1_b[l].reshape(1, D),
                       W1[l], b1[l].reshape(1, 2 * D), W2[l],
                       b2[l].reshape(1, D), bn2_g[l].reshape(1, D),
                       bn2_b[l].reshape(1, D))
    return _readout_call(h, Wr0, br0.reshape(1, D // 2), Wr1,
                         br1.reshape(1, D // 4), Wr2, br2.reshape(1, 1))


# speedup vs baseline: 30.1789x; 1.4347x over previous
import functools

import jax
import jax.numpy as jnp
from jax import lax
from jax.experimental import pallas as pl
from jax.experimental.pallas import tpu as pltpu
from jax.experimental.pallas import tpu_sc as plsc

N = 10000
E = 320000
D = 128
H = 8
DH = 16
L = 4
PAD_ATOM = 32

NC = 2
NS = 16
NW = NC * NS
EPW = E // NW
EC = 40
NCHUNK = EPW // EC
NACC = 10240
NPT = NACC // NS
NR = 40
NZ = NACC // 8
NZPT = NZ // NS
INV_SQRT_DH = 1.0 / (DH ** 0.5)
_GROUPS = ((0, 0), (16, 0), (24, 8))
_PERM = tuple((j % 8) * DH + 2 * i + (j // 8)
              for i in range(H) for j in range(16))


def _embed_body(nf_ref, emb_ref, out_ref):
    nf = nf_ref[...]
    oh = (jnp.broadcast_to(nf, (N, PAD_ATOM))
          == lax.broadcasted_iota(jnp.int32, (N, PAD_ATOM), 1))
    out_ref[...] = jnp.dot(oh.astype(jnp.float32), emb_ref[...],
                           preferred_element_type=jnp.float32)


_embed_call = pl.pallas_call(
    _embed_body,
    out_shape=jax.ShapeDtypeStruct((N, D), jnp.float32),
)


def _qkv_body(h_ref, wk_ref, wv_ref, wq_ref, k_ref, v_ref, q_ref):
    h = h_ref[...]
    k_ref[...] = jnp.dot(h, wk_ref[...], preferred_element_type=jnp.float32)
    v_ref[...] = jnp.dot(h, wv_ref[...], preferred_element_type=jnp.float32)
    q_ref[...] = jnp.dot(h, wq_ref[...], preferred_element_type=jnp.float32)


_qkv_call = pl.pallas_call(
    _qkv_body,
    out_shape=(jax.ShapeDtypeStruct((N, D), jnp.float32),
               jax.ShapeDtypeStruct((N, D), jnp.float32),
               jax.ShapeDtypeStruct((N, D), jnp.float32)),
)


EPB = 8000


def _ep_body(ef_ref, m_ref, b_ref, out_ref):
    out_ref[...] = jnp.dot(ef_ref[...], m_ref[...],
                           preferred_element_type=jnp.float32) + b_ref[...]


_ep_call = pl.pallas_call(
    _ep_body,
    grid=(E // EPB,),
    in_specs=[pl.BlockSpec((EPB, 16), lambda i: (i, 0)),
              pl.BlockSpec((16, D), lambda i: (0, 0)),
              pl.BlockSpec((1, D), lambda i: (0, 0))],
    out_specs=pl.BlockSpec((EPB, D), lambda i: (i, 0)),
    out_shape=jax.ShapeDtypeStruct((E, D), jnp.float32),
)


def _tail_body(acc_ref, z_ref, h_ref, wo_ref, bo_ref, g1_ref, b1_ref, w1_ref,
               bb1_ref, w2_ref, bb2_ref, g2_ref, b2_ref, out_ref):
    wv = acc_ref[0, :N] + acc_ref[1, :N]
    z = z_ref[0] + z_ref[1]
    zinv = 1.0 / (z + 1e-6)
    row = lax.broadcasted_iota(jnp.int32, (H, D), 0)
    col = lax.broadcasted_iota(jnp.int32, (H, D), 1) % H
    ex = (row == col).astype(jnp.float32)
    attn = wv * jnp.dot(zinv, ex, preferred_element_type=jnp.float32)
    h_att = jnp.dot(attn, wo_ref[...],
                    preferred_element_type=jnp.float32) + bo_ref[...]
    h1 = h_ref[...] + h_att
    mu = jnp.mean(h1, axis=0, keepdims=True)
    var = jnp.mean((h1 - mu) ** 2, axis=0, keepdims=True)
    h2 = (h1 - mu) * lax.rsqrt(var + 1e-5) * g1_ref[...] + b1_ref[...]
    r = jnp.maximum(jnp.dot(h2, w1_ref[...],
                            preferred_element_type=jnp.float32) + bb1_ref[...],
                    0.0)
    h3 = h2 + jnp.dot(r, w2_ref[...],
                      preferred_element_type=jnp.float32) + bb2_ref[...]
    mu2 = jnp.mean(h3, axis=0, keepdims=True)
    var2 = jnp.mean((h3 - mu2) ** 2, axis=0, keepdims=True)
    out_ref[...] = ((h3 - mu2) * lax.rsqrt(var2 + 1e-5) * g2_ref[...]
                    + b2_ref[...])


_tail_call = pl.pallas_call(
    _tail_body,
    out_shape=jax.ShapeDtypeStruct((N, D), jnp.float32),
)


def _readout_body(h_ref, w0_ref, b0_ref, w1_ref, b1_ref, w2_ref, b2_ref,
                  out_ref):
    x = jnp.maximum(jnp.dot(h_ref[...], w0_ref[...],
                            preferred_element_type=jnp.float32) + b0_ref[...],
                    0.0)
    x = jnp.maximum(jnp.dot(x, w1_ref[...],
                            preferred_element_type=jnp.float32) + b1_ref[...],
                    0.0)
    out_ref[...] = jnp.dot(x, w2_ref[...],
                           preferred_element_type=jnp.float32) + b2_ref[...]


_readout_call = pl.pallas_call(
    _readout_body,
    out_shape=jax.ShapeDtypeStruct((N, 1), jnp.float32),
)


DUMP = NACC - 1
DUMPZ = NZ - 1


def _edge_body(k_hbm, v_hbm, q_hbm, src_hbm, dst_hbm, ep_hbm,
               out_hbm, outz_hbm,
               sidx2, didx2, kb2, qb2, eb, pb, pbz,
               acc, accz, semk0, semk1, semq0, semq1, semv, semeb, semsc,
               semi):
    cid = lax.axis_index("c")
    sid = lax.axis_index("s")
    wid = cid * NS + sid
    z16 = jnp.zeros((DH,), jnp.float32)
    iota16 = lax.iota(jnp.int32, DH)
    idx_swap = ((iota16 + 8) & 15).astype(jnp.int32)
    zmask = iota16 < 8
    dumpv = jnp.full((DH,), DUMP, jnp.int32)
    dumpz = jnp.full((DH,), DUMPZ, jnp.int32)
    ebase = wid * EPW

    def zinit_body(e, _):
        for g in range(8):
            pb[e, pl.ds(g * DH, DH)] = z16
            pbz[e, pl.ds(g * DH, DH)] = z16
        return 0

    lax.fori_loop(0, EC, zinit_body, 0)
    base_row = sid * NPT
    for r in range(NPT // EC):
        pltpu.sync_copy(pb, acc.at[pl.ds(base_row + r * EC, EC)])
    for r in range(NZPT // EC):
        pltpu.sync_copy(pb, accz.at[pl.ds(sid * NZPT + r * EC, EC)])
    plsc.subcore_barrier()

    def idx_off(j):
        return jnp.minimum(ebase + j * EC, E - EC)

    semk = {0: semk0, 1: semk1}
    semq = {0: semq0, 1: semq1}

    pltpu.sync_copy(src_hbm.at[pl.ds(idx_off(0), EC)], sidx2.at[0])
    pltpu.sync_copy(dst_hbm.at[pl.ds(idx_off(0), EC)], didx2.at[0])
    pltpu.sync_copy(src_hbm.at[pl.ds(idx_off(1), EC)], sidx2.at[1])
    pltpu.sync_copy(dst_hbm.at[pl.ds(idx_off(1), EC)], didx2.at[1])
    pltpu.async_copy(k_hbm.at[sidx2.at[0]], kb2.at[0], semk[0])
    pltpu.async_copy(q_hbm.at[didx2.at[0]], qb2.at[0], semq[0])
    pltpu.async_copy(ep_hbm.at[pl.ds(idx_off(0), EC)], eb, semeb)

    def scat_drain():
        for _ in range(3):
            pltpu.make_async_copy(pb.at[pl.ds(0, DH)], acc.at[dumpv],
                                  semsc).wait()
            pltpu.make_async_copy(pbz.at[pl.ds(0, DH)], accz.at[dumpz],
                                  semsc).wait()

    def idx_drain(p):
        pltpu.make_async_copy(src_hbm.at[pl.ds(idx_off(0), EC)],
                              sidx2.at[p], semi).wait()
        pltpu.make_async_copy(dst_hbm.at[pl.ds(idx_off(0), EC)],
                              didx2.at[p], semi).wait()

    def half_body(p, m, offv_prev):
        j = 2 * m + p
        first = (m == 0)

        if p == 0:
            @pl.when(jnp.logical_not(first))
            def _():
                scat_drain()
        else:
            scat_drain()

        for gi, (g0, l0) in enumerate(_GROUPS):
            for i in range(l0, DH):
                pbz[g0 + i, pl.ds(offv_prev[gi][i], DH)] = z16

        pltpu.make_async_copy(k_hbm.at[sidx2.at[p]], kb2.at[p],
                              semk[p]).wait()
        pltpu.make_async_copy(q_hbm.at[didx2.at[p]], qb2.at[p],
                              semq[p]).wait()
        pltpu.make_async_copy(ep_hbm.at[pl.ds(idx_off(0), EC)], eb,
                              semeb).wait()

        pltpu.async_copy(v_hbm.at[sidx2.at[p]], pb, semv)

        dvs, offv, zoffs = [], [], {}
        for gi, (g0, l0) in enumerate(_GROUPS):
            dv = didx2[p, pl.ds(g0, DH)]
            dvs.append(dv)
            ov = (dv & 7) * DH
            offv.append(ov)
            for i in range(l0, DH):
                zoffs[g0 + i] = ov[i]
        i_sc = [dvs[0], dvs[1], jnp.where(zmask, dumpv, dvs[2])]
        i_z = [lax.shift_right_logical(dvs[0], 3),
               lax.shift_right_logical(dvs[1], 3),
               jnp.where(zmask, dumpz, lax.shift_right_logical(dvs[2], 3))]

        for e in range(EC):
            ts = []
            for b in range(H):
                kv_ = kb2[p, e, pl.ds(b * DH, DH)]
                qv_ = qb2[p, e, pl.ds(b * DH, DH)]
                ev_ = eb[e, pl.ds(b * DH, DH)]
                ts.append(kv_ * qv_ * ev_)
            while len(ts) > 1:
                ts = [ts[i] + ts[i + 1] for i in range(0, len(ts), 2)]
            t = ts[0]
            s = (t + t[idx_swap]) * INV_SQRT_DH
            s = jnp.exp(jnp.minimum(jnp.maximum(s, -5.0), 5.0))
            eb[e, pl.ds(0, DH)] = s
            pbz[e, pl.ds(zoffs[e], DH)] = jnp.where(zmask, s, 0.0)

        if p == 0:
            @pl.when(jnp.logical_not(first))
            def _():
                idx_drain(1)
        else:
            idx_drain(0)
        pltpu.async_copy(k_hbm.at[sidx2.at[1 - p]], kb2.at[1 - p],
                         semk[1 - p])
        pltpu.async_copy(q_hbm.at[didx2.at[1 - p]], qb2.at[1 - p],
                         semq[1 - p])

        pltpu.make_async_copy(v_hbm.at[sidx2.at[p]], pb, semv).wait()
        for e in range(EC):
            s = eb[e, pl.ds(0, DH)]
            for b in range(H):
                pb[e, pl.ds(b * DH, DH)] = pb[e, pl.ds(b * DH, DH)] * s

        pltpu.async_copy(ep_hbm.at[pl.ds(idx_off(j + 1), EC)], eb, semeb)

        pltpu.async_copy(src_hbm.at[pl.ds(idx_off(j + 2), EC)],
                         sidx2.at[p], semi)
        pltpu.async_copy(dst_hbm.at[pl.ds(idx_off(j + 2), EC)],
                         didx2.at[p], semi)

        for gsl, iv in zip((0, 16, 24), i_sc):
            pltpu.async_copy(pb.at[pl.ds(gsl, DH)], acc.at[iv], semsc,
                             add=True)
        for gsl, iv in zip((0, 16, 24), i_z):
            pltpu.async_copy(pbz.at[pl.ds(gsl, DH)], accz.at[iv], semsc,
                             add=True)
        return offv

    def pair_body(m, carry):
        offv = half_body(0, m, carry)
        offv = half_body(1, m, offv)
        return offv

    zero_i = jnp.zeros((DH,), jnp.int32)
    lax.fori_loop(0, NCHUNK // 2, pair_body,
                  [zero_i, zero_i, zero_i])

    scat_drain()
    pltpu.make_async_copy(k_hbm.at[sidx2.at[0]], kb2.at[0], semk[0]).wait()
    pltpu.make_async_copy(q_hbm.at[didx2.at[0]], qb2.at[0], semq[0]).wait()
    pltpu.make_async_copy(ep_hbm.at[pl.ds(idx_off(0), EC)], eb, semeb).wait()
    idx_drain(1)
    plsc.subcore_barrier()

    for r in range(NPT // EC):
        sl = pl.ds(base_row + r * EC, EC)
        pltpu.sync_copy(acc.at[sl], pb)
        pltpu.sync_copy(pb, out_hbm.at[cid, sl])
    for r in range(NZPT // EC):
        zsl = pl.ds(sid * NZPT + r * EC, EC)
        pltpu.sync_copy(accz.at[zsl], pb)
        pltpu.sync_copy(pb, outz_hbm.at[cid, zsl])


@functools.lru_cache(maxsize=1)
def _edge_call():
    return pl.kernel(
        _edge_body,
        out_type=(jax.ShapeDtypeStruct((NC, NACC, D), jnp.float32),
                  jax.ShapeDtypeStruct((NC, NZ, D), jnp.float32)),
        mesh=plsc.VectorSubcoreMesh(core_axis_name="c", subcore_axis_name="s",
                                    num_cores=NC, num_subcores=NS),
        compiler_params=pltpu.CompilerParams(needs_layout_passes=False),
        scratch_types=[
            pltpu.VMEM((2, EC), jnp.int32),
            pltpu.VMEM((2, EC), jnp.int32),
            pltpu.VMEM((2, EC, D), jnp.float32),
            pltpu.VMEM((2, EC, D), jnp.float32),
            pltpu.VMEM((EC, D), jnp.float32),
            pltpu.VMEM((EC, D), jnp.float32),
            pltpu.VMEM((EC, D), jnp.float32),
            pltpu.VMEM_SHARED((NACC, D), jnp.float32),
            pltpu.VMEM_SHARED((NZ, D), jnp.float32),
            pltpu.SemaphoreType.DMA,
            pltpu.SemaphoreType.DMA,
            pltpu.SemaphoreType.DMA,
            pltpu.SemaphoreType.DMA,
            pltpu.SemaphoreType.DMA,
            pltpu.SemaphoreType.DMA,
            pltpu.SemaphoreType.DMA,
            pltpu.SemaphoreType.DMA,
        ],
    )


def kernel(node_feat, edge_index, edge_feat, p, emb_h, We_e, be_e, Wq, Wk,
           Wv, Wpe, Wo, bo, bn1_g, bn1_b, W1, b1, W2, b2, bn2_g, bn2_b,
           Wr0, br0, Wr1, br1, Wr2, br2):
    nf = node_feat.astype(jnp.int32).reshape(N, 1)
    ei = edge_index.astype(jnp.int32)
    emb_pad = jnp.zeros((PAD_ATOM, D), jnp.float32).at[:emb_h.shape[0]].set(emb_h)
    ef16 = jnp.zeros((E, 16), jnp.float32).at[:, :4].set(edge_feat)

    perm = jnp.array(_PERM, jnp.int32)
    h = _embed_call(nf, emb_pad)
    for l in range(L):
        k, v, q = _qkv_call(h, Wk[l][:, perm], Wv[l][:, perm], Wq[l][:, perm])
        m16 = jnp.zeros((16, D), jnp.float32).at[:4].set(We_e @ Wpe[l])
        ept = _ep_call(ef16, m16[:, perm], (be_e @ Wpe[l])[perm].reshape(1, D))
        acc2, accz = _edge_call()(k, v, q, ei[0], ei[1], ept)
        z2 = accz.reshape(NC, NZ, 8, 16)[:, :, :, :H].reshape(NC, NACC, H)[:, :N, :]
        h = _tail_call(acc2, z2, h, Wo[l][perm, :], bo[l].reshape(1, D),
                       bn1_g[l].reshape(1, D), bn1_b[l].reshape(1, D),
                       W1[l], b1[l].reshape(1, 2 * D), W2[l],
                       b2[l].reshape(1, D), bn2_g[l].reshape(1, D),
                       bn2_b[l].reshape(1, D))
    return _readout_call(h, Wr0, br0.reshape(1, D // 2), Wr1,
                         br1.reshape(1, D // 4), Wr2, br2.reshape(1, 1))
